# baseline (speedup 1.0000x reference)
import functools

import jax
import jax.numpy as jnp
import numpy as np
from jax import lax
from jax.experimental import pallas as pl
from jax.experimental.pallas import tpu as pltpu
from jax.experimental.pallas import tpu_sc as plsc

_DIM = 32
_NCODES = 8192
_ROWS = 8192
_SIGMA = float(np.sqrt(max(0.001, 1e-12)))
_RB = 1024
_CB = 2048


def _round_to_bf16_f32(x):
    u = lax.bitcast_convert_type(x, jnp.uint32)
    r = u + jnp.uint32(0x7FFF) + ((u >> jnp.uint32(16)) & jnp.uint32(1))
    return lax.bitcast_convert_type(r & jnp.uint32(0xFFFF0000), jnp.float32)


def _argmin_body(z_ref, zb_ref, cb_ref, cb16_ref, idx_ref):
    z = z_ref[...]
    znorm = jnp.sum(z * z, axis=1)
    zb = zb_ref[...]

    def chunk(c, carry):
        dmin, imin = carry
        cb = cb_ref[pl.ds(c * _CB, _CB), :]
        cnorm = jnp.sum(cb * cb, axis=1)
        prod = lax.dot_general(
            zb, cb16_ref[pl.ds(c * _CB, _CB), :], (((1,), (1,)), ((), ())),
            preferred_element_type=jnp.float32,
        )
        d = znorm[:, None] - 2.0 * prod + cnorm[None, :]
        dchunk = jnp.min(d, axis=1)
        iota = lax.broadcasted_iota(jnp.int32, d.shape, 1)
        ichunk = jnp.min(
            jnp.where(d == dchunk[:, None], iota, jnp.int32(2**30)), axis=1
        ) + c * _CB
        upd = dchunk < dmin
        return (jnp.where(upd, _round_to_bf16_f32(dchunk), dmin),
                jnp.where(upd, ichunk, imin))

    dmin0 = jnp.full((_RB,), jnp.inf, jnp.float32)
    imin0 = jnp.zeros((_RB,), jnp.int32)
    _, imin = lax.fori_loop(0, _NCODES // _CB, chunk, (dmin0, imin0))
    idx_ref[...] = imin


def _nearest_code(z_flat, codebook):
    return pl.pallas_call(
        _argmin_body,
        grid=(_ROWS // _RB,),
        in_specs=[
            pl.BlockSpec((_RB, _DIM), lambda i: (i, 0)),
            pl.BlockSpec((_RB, _DIM), lambda i: (i, 0)),
            pl.BlockSpec((_NCODES, _DIM), lambda i: (0, 0)),
            pl.BlockSpec((_NCODES, _DIM), lambda i: (0, 0)),
        ],
        out_specs=pl.BlockSpec((_RB,), lambda i: (i,)),
        out_shape=jax.ShapeDtypeStruct((_ROWS,), jnp.int32),
    )(z_flat, z_flat.astype(jnp.bfloat16), codebook,
      codebook.astype(jnp.bfloat16))


def _sc_gather(indices, codebook, ema_probs):
    info = plsc.get_sparse_core_info()
    nw = info.num_cores * info.num_subcores
    bpw = _ROWS // nw
    nj = bpw // 128
    mesh = plsc.VectorSubcoreMesh(core_axis_name="c", subcore_axis_name="s")

    @functools.partial(
        pl.kernel,
        mesh=mesh,
        compiler_params=pltpu.CompilerParams(use_tc_tiling_on_sc=False),
        out_type=[
            jax.ShapeDtypeStruct((_ROWS, _DIM), jnp.float32),
            jax.ShapeDtypeStruct((_ROWS, 16), jnp.float32),
        ],
        scratch_types=[
            pltpu.VMEM((nj, 128), jnp.int32),
            pltpu.VMEM((bpw, _DIM), jnp.float32),
            pltpu.VMEM((bpw, 16), jnp.float32),
            pltpu.SemaphoreType.DMA,
        ],
    )
    def gather(idx_hbm, cb_hbm, ema_hbm, hard_hbm, probs_hbm,
               idx_v, rows_v, probs_v, sem):
        wid = lax.axis_index("s") * info.num_cores + lax.axis_index("c")
        base = wid * bpw
        pltpu.sync_copy(idx_hbm.at[pl.ds(wid * nj, nj)], idx_v)
        cps = []
        for j in range(nj):
            cps.append(pltpu.async_copy(
                cb_hbm.at[idx_v.at[j]], rows_v.at[pl.ds(j * 128, 128)], sem))
            cps.append(pltpu.async_copy(
                ema_hbm.at[idx_v.at[j]], probs_v.at[pl.ds(j * 128, 128)], sem))
        for cp in cps:
            cp.wait()
        pltpu.sync_copy(rows_v, hard_hbm.at[pl.ds(base, bpw)])
        pltpu.sync_copy(probs_v, probs_hbm.at[pl.ds(base, bpw)])

    ema16 = jnp.broadcast_to(ema_probs[:, None], (_NCODES, 16))
    hard, probs16 = gather(
        indices.reshape(_ROWS // 128, 128), codebook, ema16
    )
    return hard, probs16[:, 0]


def _finish_body(z_ref, hard_ref, probs_ref, ema_ref,
                 q_ref, pclip_ref, logp_ref, perp_ref):
    z = z_ref[...]
    h = hard_ref[...]
    diff = z - h
    nrm = jnp.sqrt(jnp.sum(diff * diff, axis=1, keepdims=True))
    nrm = jnp.maximum(nrm, jnp.float32(1e-12))
    approx = h + _SIGMA * diff / nrm
    q_ref[...] = approx + (h - approx)
    p = jnp.maximum(probs_ref[...], jnp.float32(1e-9))
    pclip_ref[...] = p
    logp_ref[...] = jnp.log(p)
    pe = jnp.maximum(ema_ref[...], jnp.float32(1e-9))
    perp_ref[...] = jnp.broadcast_to(jnp.exp(-jnp.sum(pe * jnp.log(pe))), (1, 1))


def _finish(z_flat, hard, probs, ema_probs):
    return pl.pallas_call(
        _finish_body,
        out_shape=[
            jax.ShapeDtypeStruct((_ROWS, _DIM), jnp.float32),
            jax.ShapeDtypeStruct((_ROWS,), jnp.float32),
            jax.ShapeDtypeStruct((_ROWS,), jnp.float32),
            jax.ShapeDtypeStruct((1, 1), jnp.float32),
        ],
    )(z_flat, hard, probs, ema_probs)


def kernel(z, codebook, ema_probs):
    n, c, h, w = z.shape
    z_flat = jnp.transpose(z, (0, 2, 3, 1)).reshape(-1, c)
    indices = _nearest_code(z_flat, codebook)
    hard, probs = _sc_gather(indices, codebook, ema_probs)
    q_flat, pclip, logp, perp = _finish(z_flat, hard, probs, ema_probs)
    quantized = jnp.transpose(q_flat.reshape(n, h, w, c), (0, 3, 1, 2))
    hard_spatial = jnp.transpose(hard.reshape(n, h, w, c), (0, 3, 1, 2))
    return (
        quantized,
        hard_spatial,
        indices.reshape(n, h, w),
        pclip.reshape(n, h, w),
        logp.reshape(n, h, w),
        perp.reshape(()),
    )


# speedup vs baseline: 1.0326x; 1.0326x over previous
import functools

import jax
import jax.numpy as jnp
import numpy as np
from jax import lax
from jax.experimental import pallas as pl
from jax.experimental.pallas import tpu as pltpu
from jax.experimental.pallas import tpu_sc as plsc

_DIM = 32
_NCODES = 8192
_ROWS = 8192
_RB = 1024
_CB = 2048


def _round_to_bf16_f32(x):
    u = lax.bitcast_convert_type(x, jnp.uint32)
    r = u + jnp.uint32(0x7FFF) + ((u >> jnp.uint32(16)) & jnp.uint32(1))
    return lax.bitcast_convert_type(r & jnp.uint32(0xFFFF0000), jnp.float32)


def _argmin_body(z_ref, zb_ref, cb_ref, cb16_ref, ema_ref,
                 idx_ref, table_ref, perp_ref):
    z = z_ref[...]
    znorm = jnp.sum(z * z, axis=1)
    zb = zb_ref[...]

    def chunk(c, carry):
        dmin, imin = carry
        cb = cb_ref[pl.ds(c * _CB, _CB), :]
        cnorm = jnp.sum(cb * cb, axis=1)
        prod = lax.dot_general(
            zb, cb16_ref[pl.ds(c * _CB, _CB), :], (((1,), (1,)), ((), ())),
            preferred_element_type=jnp.float32,
        )
        d = znorm[:, None] - 2.0 * prod + cnorm[None, :]
        dchunk = jnp.min(d, axis=1)
        iota = lax.broadcasted_iota(jnp.int32, d.shape, 1)
        ichunk = jnp.min(
            jnp.where(d == dchunk[:, None], iota, jnp.int32(2**30)), axis=1
        ) + c * _CB
        upd = dchunk < dmin
        return (jnp.where(upd, _round_to_bf16_f32(dchunk), dmin),
                jnp.where(upd, ichunk, imin))

    dmin0 = jnp.full((_RB,), jnp.inf, jnp.float32)
    imin0 = jnp.zeros((_RB,), jnp.int32)
    _, imin = lax.fori_loop(0, _NCODES // _CB, chunk, (dmin0, imin0))
    idx_ref[...] = imin

    @pl.when(pl.program_id(0) == 0)
    def _():
        pe = jnp.maximum(ema_ref[...], jnp.float32(1e-9))
        logpe = jnp.log(pe)
        col = lax.broadcasted_iota(jnp.int32, (_NCODES, 16), 1)
        table_ref[...] = jnp.where(
            col == 0,
            jnp.broadcast_to(pe, (_NCODES, 16)),
            jnp.broadcast_to(logpe, (_NCODES, 16)),
        )
        perp_ref[...] = jnp.broadcast_to(jnp.exp(-jnp.sum(pe * logpe)), (1, 1))


def _nearest_code(z_flat, codebook, ema_probs):
    return pl.pallas_call(
        _argmin_body,
        grid=(_ROWS // _RB,),
        in_specs=[
            pl.BlockSpec((_RB, _DIM), lambda i: (i, 0)),
            pl.BlockSpec((_RB, _DIM), lambda i: (i, 0)),
            pl.BlockSpec((_NCODES, _DIM), lambda i: (0, 0)),
            pl.BlockSpec((_NCODES, _DIM), lambda i: (0, 0)),
            pl.BlockSpec((_NCODES, 1), lambda i: (0, 0)),
        ],
        out_specs=[
            pl.BlockSpec((_RB,), lambda i: (i,)),
            pl.BlockSpec((_NCODES, 16), lambda i: (0, 0)),
            pl.BlockSpec((1, 1), lambda i: (0, 0)),
        ],
        out_shape=[
            jax.ShapeDtypeStruct((_ROWS,), jnp.int32),
            jax.ShapeDtypeStruct((_NCODES, 16), jnp.float32),
            jax.ShapeDtypeStruct((1, 1), jnp.float32),
        ],
    )(z_flat, z_flat.astype(jnp.bfloat16), codebook,
      codebook.astype(jnp.bfloat16), ema_probs.reshape(_NCODES, 1))


def _sc_gather(indices, codebook, table):
    info = plsc.get_sparse_core_info()
    nw = info.num_cores * info.num_subcores
    bpw = _ROWS // nw
    nj = bpw // 128
    mesh = plsc.VectorSubcoreMesh(core_axis_name="c", subcore_axis_name="s")

    @functools.partial(
        pl.kernel,
        mesh=mesh,
        compiler_params=pltpu.CompilerParams(use_tc_tiling_on_sc=False),
        out_type=[
            jax.ShapeDtypeStruct((_ROWS, _DIM), jnp.float32),
            jax.ShapeDtypeStruct((_ROWS, 16), jnp.float32),
        ],
        scratch_types=[
            pltpu.VMEM((nj, 128), jnp.int32),
            pltpu.VMEM((bpw, _DIM), jnp.float32),
            pltpu.VMEM((bpw, 16), jnp.float32),
            pltpu.SemaphoreType.DMA,
        ],
    )
    def gather(idx_hbm, cb_hbm, tab_hbm, hard_hbm, probs_hbm,
               idx_v, rows_v, probs_v, sem):
        wid = lax.axis_index("s") * info.num_cores + lax.axis_index("c")
        base = wid * bpw
        pltpu.sync_copy(idx_hbm.at[pl.ds(wid * nj, nj)], idx_v)
        cps = []
        for j in range(nj):
            cps.append(pltpu.async_copy(
                cb_hbm.at[idx_v.at[j]], rows_v.at[pl.ds(j * 128, 128)], sem))
            cps.append(pltpu.async_copy(
                tab_hbm.at[idx_v.at[j]], probs_v.at[pl.ds(j * 128, 128)], sem))
        for cp in cps:
            cp.wait()
        pltpu.sync_copy(rows_v, hard_hbm.at[pl.ds(base, bpw)])
        pltpu.sync_copy(probs_v, probs_hbm.at[pl.ds(base, bpw)])

    hard, out16 = gather(indices.reshape(_ROWS // 128, 128), codebook, table)
    return hard, out16[:, 0], out16[:, 1]


def kernel(z, codebook, ema_probs):
    n, c, h, w = z.shape
    z_flat = jnp.transpose(z, (0, 2, 3, 1)).reshape(-1, c)
    indices, table, perp = _nearest_code(z_flat, codebook, ema_probs)
    hard, pclip, logp = _sc_gather(indices, codebook, table)
    hard_spatial = jnp.transpose(hard.reshape(n, h, w, c), (0, 3, 1, 2))
    return (
        hard_spatial,
        hard_spatial,
        indices.reshape(n, h, w),
        pclip.reshape(n, h, w),
        logp.reshape(n, h, w),
        perp.reshape(()),
    )
